# baseline (speedup 1.0000x reference)
import functools

import jax
import jax.numpy as jnp
from jax import lax
from jax.experimental import pallas as pl
from jax.experimental.pallas import tpu as pltpu
from jax.experimental.pallas import tpu_sc as plsc

N = 10000
E = 320000
NC = 2
NS = 16
NW = NC * NS
CHUNK = 128
NCHUNK = 80
NCTOT = NCHUNK + 2
NACC = 10240
ZROWS = NACC // NS
DDEG = 16

_MESH = plsc.VectorSubcoreMesh(core_axis_name="c", subcore_axis_name="s")
_SC_PARAMS = pltpu.CompilerParams(use_tc_tiling_on_sc=False)


def _make_deg():
    @functools.partial(
        pl.kernel,
        out_type=jax.ShapeDtypeStruct((NC, NACC, DDEG), jnp.float32),
        mesh=_MESH,
        compiler_params=_SC_PARAMS,
        scratch_types=[
            pltpu.VMEM((NCTOT, CHUNK), jnp.int32),
            pltpu.VMEM((CHUNK, DDEG), jnp.float32),
            pltpu.VMEM((ZROWS, DDEG), jnp.float32),
            pltpu.VMEM_SHARED((NACC, DDEG), jnp.float32),
            pltpu.SemaphoreType.DMA,
        ],
    )
    def deg(dst_hbm, ones_hbm, zeros_hbm, out_hbm, dst_v, ones_v, zbuf_v, acc_sh, sem):
        cid = lax.axis_index("c")
        sid = lax.axis_index("s")
        wid = sid * NC + cid
        pltpu.sync_copy(zeros_hbm, zbuf_v)
        pltpu.sync_copy(zbuf_v, acc_sh.at[pl.ds(sid * ZROWS, ZROWS)])
        pltpu.sync_copy(ones_hbm, ones_v)
        pltpu.sync_copy(dst_hbm.at[wid], dst_v)
        plsc.subcore_barrier()

        def body(j, carry):
            pltpu.sync_copy(ones_v, acc_sh.at[dst_v.at[j]], add=True)
            return carry

        lax.fori_loop(0, NCHUNK, body, 0)
        plsc.subcore_barrier()
        pltpu.sync_copy(acc_sh.at[pl.ds(sid * ZROWS, ZROWS)], zbuf_v)
        pltpu.sync_copy(zbuf_v, out_hbm.at[cid, pl.ds(sid * ZROWS, ZROWS)])

    return deg


def _make_agg(d):
    @functools.partial(
        pl.kernel,
        out_type=jax.ShapeDtypeStruct((NC, NACC, d), jnp.float32),
        mesh=_MESH,
        compiler_params=_SC_PARAMS,
        scratch_types=[
            pltpu.VMEM((NCTOT, CHUNK), jnp.int32),
            pltpu.VMEM((NCTOT, CHUNK), jnp.int32),
            pltpu.VMEM((CHUNK, d), jnp.float32),
            pltpu.VMEM((CHUNK, d), jnp.float32),
            pltpu.VMEM((ZROWS, d), jnp.float32),
            pltpu.VMEM_SHARED((NACC, d), jnp.float32),
            pltpu.SemaphoreType.DMA,
            pltpu.SemaphoreType.DMA,
        ],
    )
    def agg(hp_hbm, src_hbm, dst_hbm, zeros_hbm, out_hbm,
            src_v, dst_v, rows0, rows1, zbuf_v, acc_sh, sem0, sem1):
        cid = lax.axis_index("c")
        sid = lax.axis_index("s")
        wid = sid * NC + cid
        pltpu.sync_copy(zeros_hbm, zbuf_v)
        pltpu.sync_copy(zbuf_v, acc_sh.at[pl.ds(sid * ZROWS, ZROWS)])
        pltpu.sync_copy(src_hbm.at[wid], src_v)
        pltpu.sync_copy(dst_hbm.at[wid], dst_v)
        plsc.subcore_barrier()

        pltpu.async_copy(hp_hbm.at[src_v.at[0]], rows0, sem0)
        pltpu.async_copy(hp_hbm.at[src_v.at[1]], rows1, sem1)

        def body(i, carry):
            j = 2 * i
            pltpu.make_async_copy(hp_hbm.at[src_v.at[j]], rows0, sem0).wait()
            pltpu.sync_copy(rows0, acc_sh.at[dst_v.at[j]], add=True)
            pltpu.async_copy(hp_hbm.at[src_v.at[j + 2]], rows0, sem0)
            pltpu.make_async_copy(hp_hbm.at[src_v.at[j + 1]], rows1, sem1).wait()
            pltpu.sync_copy(rows1, acc_sh.at[dst_v.at[j + 1]], add=True)
            pltpu.async_copy(hp_hbm.at[src_v.at[j + 3]], rows1, sem1)
            return carry

        lax.fori_loop(0, NCHUNK // 2, body, 0)
        pltpu.make_async_copy(hp_hbm.at[src_v.at[NCHUNK]], rows0, sem0).wait()
        pltpu.make_async_copy(hp_hbm.at[src_v.at[NCHUNK + 1]], rows1, sem1).wait()

        plsc.subcore_barrier()
        pltpu.sync_copy(acc_sh.at[pl.ds(sid * ZROWS, ZROWS)], zbuf_v)
        pltpu.sync_copy(zbuf_v, out_hbm.at[cid, pl.ds(sid * ZROWS, ZROWS)])

    return agg


_deg_call = _make_deg()
_agg64 = _make_agg(64)
_agg32 = _make_agg(32)

_RB = 2000
_GRID = N // _RB


def _mm_body(x_ref, w_ref, o_ref):
    o_ref[...] = jnp.dot(x_ref[...], w_ref[...],
                         preferred_element_type=jnp.float32)


def _tc_matmul(x, w):
    k, m = x.shape[1], w.shape[1]
    return pl.pallas_call(
        _mm_body,
        grid=(_GRID,),
        in_specs=[
            pl.BlockSpec((_RB, k), lambda i: (i, 0)),
            pl.BlockSpec((k, m), lambda i: (0, 0)),
        ],
        out_specs=pl.BlockSpec((_RB, m), lambda i: (i, 0)),
        out_shape=jax.ShapeDtypeStruct((N, m), jnp.float32),
    )(x, w)


def _scale_body(degp_ref, h_ref, o_ref):
    deg = degp_ref[0, :, 0:1] + degp_ref[1, :, 0:1] + 1.0
    o_ref[...] = h_ref[...] * lax.rsqrt(deg)


def _tc_scale(degp, h):
    m = h.shape[1]
    return pl.pallas_call(
        _scale_body,
        grid=(_GRID,),
        in_specs=[
            pl.BlockSpec((NC, _RB, DDEG), lambda i: (0, i, 0)),
            pl.BlockSpec((_RB, m), lambda i: (i, 0)),
        ],
        out_specs=pl.BlockSpec((_RB, m), lambda i: (i, 0)),
        out_shape=jax.ShapeDtypeStruct((N, m), jnp.float32),
    )(degp, h)


def _mid_body(degp_ref, hp_ref, aggp_ref, b_ref, w_ref, o_ref):
    deg = degp_ref[0, :, 0:1] + degp_ref[1, :, 0:1] + 1.0
    dis = lax.rsqrt(deg)
    tot = hp_ref[...] + aggp_ref[0] + aggp_ref[1]
    h2 = jnp.maximum(tot * dis + b_ref[...], 0.0)
    o_ref[...] = jnp.dot(h2, w_ref[...], preferred_element_type=jnp.float32) * dis


def _tc_mid(degp, hp1, aggp1, b1, w2):
    k, m = w2.shape
    return pl.pallas_call(
        _mid_body,
        grid=(_GRID,),
        in_specs=[
            pl.BlockSpec((NC, _RB, DDEG), lambda i: (0, i, 0)),
            pl.BlockSpec((_RB, k), lambda i: (i, 0)),
            pl.BlockSpec((NC, _RB, k), lambda i: (0, i, 0)),
            pl.BlockSpec((1, k), lambda i: (0, 0)),
            pl.BlockSpec((k, m), lambda i: (0, 0)),
        ],
        out_specs=pl.BlockSpec((_RB, m), lambda i: (i, 0)),
        out_shape=jax.ShapeDtypeStruct((N, m), jnp.float32),
    )(degp, hp1, aggp1, b1, w2)


def _fin_body(degp_ref, hp_ref, aggp_ref, b_ref, o_ref):
    deg = degp_ref[0, :, 0:1] + degp_ref[1, :, 0:1] + 1.0
    dis = lax.rsqrt(deg)
    tot = hp_ref[...] + aggp_ref[0] + aggp_ref[1]
    o_ref[...] = tot * dis + b_ref[...]


def _tc_fin(degp, hp2, aggp2, b2):
    m = hp2.shape[1]
    return pl.pallas_call(
        _fin_body,
        grid=(_GRID,),
        in_specs=[
            pl.BlockSpec((NC, _RB, DDEG), lambda i: (0, i, 0)),
            pl.BlockSpec((_RB, m), lambda i: (i, 0)),
            pl.BlockSpec((NC, _RB, m), lambda i: (0, i, 0)),
            pl.BlockSpec((1, m), lambda i: (0, 0)),
        ],
        out_specs=pl.BlockSpec((_RB, m), lambda i: (i, 0)),
        out_shape=jax.ShapeDtypeStruct((N, m), jnp.float32),
    )(degp, hp2, aggp2, b2)


def kernel(x, edge_index, W1, b1, W2, b2):
    src = edge_index[0].astype(jnp.int32)
    dst = edge_index[1].astype(jnp.int32)
    epad = NW * NCHUNK * CHUNK
    src_p = jnp.concatenate(
        [src, jnp.zeros((epad - E,), jnp.int32)]).reshape(NW, NCHUNK, CHUNK)
    dst_p = jnp.concatenate(
        [dst, jnp.full((epad - E,), N, jnp.int32)]).reshape(NW, NCHUNK, CHUNK)
    src_p = jnp.concatenate(
        [src_p, jnp.zeros((NW, 2, CHUNK), jnp.int32)], axis=1)
    dst_p = jnp.concatenate(
        [dst_p, jnp.full((NW, 2, CHUNK), N, jnp.int32)], axis=1)

    ones = jnp.ones((CHUNK, DDEG), jnp.float32)
    zeros_deg = jnp.zeros((ZROWS, DDEG), jnp.float32)
    zeros64 = jnp.zeros((ZROWS, 64), jnp.float32)
    zeros32 = jnp.zeros((ZROWS, 32), jnp.float32)

    degp = _deg_call(dst_p, ones, zeros_deg)
    h1 = _tc_matmul(x, W1)
    hp1 = _tc_scale(degp, h1)
    aggp1 = _agg64(hp1, src_p, dst_p, zeros64)
    hp2 = _tc_mid(degp, hp1, aggp1, b1.reshape(1, 64), W2)
    aggp2 = _agg32(hp2, src_p, dst_p, zeros32)
    return _tc_fin(degp, hp2, aggp2, b2.reshape(1, 32))


# speedup vs baseline: 2.3669x; 2.3669x over previous
import functools

import jax
import jax.numpy as jnp
from jax import lax
from jax.experimental import pallas as pl
from jax.experimental.pallas import tpu as pltpu
from jax.experimental.pallas import tpu_sc as plsc

N = 10000
E = 320000
NC = 2
NS = 16
NW = NC * NS
CHUNK = 128
NCHUNK = 80
NCTOT = NCHUNK + 2
NACC = 10240
ZROWS = NACC // NS
ZB = 160
DDEG = 16

_MESH = plsc.VectorSubcoreMesh(core_axis_name="c", subcore_axis_name="s")
_SC_PARAMS = pltpu.CompilerParams(use_tc_tiling_on_sc=False)


def _make_deg():
    @functools.partial(
        pl.kernel,
        out_type=jax.ShapeDtypeStruct((NC, NACC, DDEG), jnp.float32),
        mesh=_MESH,
        compiler_params=_SC_PARAMS,
        scratch_types=[
            pltpu.VMEM((NCTOT, CHUNK), jnp.int32),
            pltpu.VMEM((CHUNK, DDEG), jnp.float32),
            pltpu.VMEM((ZROWS, DDEG), jnp.float32),
            pltpu.VMEM_SHARED((NACC, DDEG), jnp.float32),
            pltpu.SemaphoreType.DMA,
        ],
    )
    def deg(dst_hbm, ones_hbm, zeros_hbm, out_hbm, dst_v, ones_v, zbuf_v, acc_sh, sem):
        cid = lax.axis_index("c")
        sid = lax.axis_index("s")
        wid = sid * NC + cid
        pltpu.sync_copy(zeros_hbm, zbuf_v)
        pltpu.sync_copy(zbuf_v, acc_sh.at[pl.ds(sid * ZROWS, ZROWS)])
        pltpu.sync_copy(ones_hbm, ones_v)
        pltpu.sync_copy(dst_hbm.at[wid], dst_v)
        plsc.subcore_barrier()

        def body(j, carry):
            pltpu.sync_copy(ones_v, acc_sh.at[dst_v.at[j]], add=True)
            return carry

        lax.fori_loop(0, NCHUNK, body, 0)
        plsc.subcore_barrier()
        pltpu.sync_copy(acc_sh.at[pl.ds(sid * ZROWS, ZROWS)], zbuf_v)
        pltpu.sync_copy(zbuf_v, out_hbm.at[cid, pl.ds(sid * ZROWS, ZROWS)])

    return deg


def _make_agg(d):
    @functools.partial(
        pl.kernel,
        out_type=jax.ShapeDtypeStruct((NC, NACC, d), jnp.float32),
        mesh=_MESH,
        compiler_params=_SC_PARAMS,
        scratch_types=[
            pltpu.VMEM((NCTOT, CHUNK), jnp.int32),
            pltpu.VMEM((NCTOT, CHUNK), jnp.int32),
            pltpu.VMEM((CHUNK, d), jnp.float32),
            pltpu.VMEM((ZB, d), jnp.float32),
            pltpu.VMEM_SHARED((NACC, d), jnp.float32),
            pltpu.VMEM_SHARED((NACC, d), jnp.float32),
            pltpu.SemaphoreType.DMA,
        ],
    )
    def agg(hp_hbm, src_hbm, dst_hbm, zeros_hbm, out_hbm,
            src_v, dst_v, rows_v, zbuf_v, table_sh, acc_sh, sem):
        cid = lax.axis_index("c")
        sid = lax.axis_index("s")
        wid = sid * NC + cid
        pltpu.sync_copy(hp_hbm.at[pl.ds(sid * ZROWS, ZROWS)],
                        table_sh.at[pl.ds(sid * ZROWS, ZROWS)])
        pltpu.sync_copy(zeros_hbm, zbuf_v)
        for p in range(ZROWS // ZB):
            pltpu.sync_copy(zbuf_v, acc_sh.at[pl.ds(sid * ZROWS + p * ZB, ZB)])
        pltpu.sync_copy(src_hbm.at[wid], src_v)
        pltpu.sync_copy(dst_hbm.at[wid], dst_v)
        plsc.subcore_barrier()

        def body(j, carry):
            pltpu.async_copy(table_sh.at[src_v.at[j]], rows_v, sem).wait()
            pltpu.sync_copy(rows_v, acc_sh.at[dst_v.at[j]], add=True)
            return carry

        lax.fori_loop(0, NCHUNK, body, 0)
        plsc.subcore_barrier()
        for p in range(ZROWS // ZB):
            pltpu.sync_copy(acc_sh.at[pl.ds(sid * ZROWS + p * ZB, ZB)], zbuf_v)
            pltpu.sync_copy(zbuf_v,
                            out_hbm.at[cid, pl.ds(sid * ZROWS + p * ZB, ZB)])

    return agg


_deg_call = _make_deg()
_agg64 = _make_agg(64)
_agg32 = _make_agg(32)

_RB = 2000
_GRID = N // _RB
_RBP = 1280
_GRIDP = NACC // _RBP


def _mm_body(x_ref, w_ref, o_ref):
    o_ref[...] = jnp.dot(x_ref[...], w_ref[...],
                         preferred_element_type=jnp.float32)


def _tc_matmul(x, w):
    k, m = x.shape[1], w.shape[1]
    return pl.pallas_call(
        _mm_body,
        grid=(_GRIDP,),
        in_specs=[
            pl.BlockSpec((_RBP, k), lambda i: (i, 0)),
            pl.BlockSpec((k, m), lambda i: (0, 0)),
        ],
        out_specs=pl.BlockSpec((_RBP, m), lambda i: (i, 0)),
        out_shape=jax.ShapeDtypeStruct((NACC, m), jnp.float32),
    )(x, w)


def _scale_body(degp_ref, h_ref, o_ref):
    deg = degp_ref[0, :, 0:1] + degp_ref[1, :, 0:1] + 1.0
    o_ref[...] = h_ref[...] * lax.rsqrt(deg)


def _tc_scale(degp, h):
    m = h.shape[1]
    return pl.pallas_call(
        _scale_body,
        grid=(_GRIDP,),
        in_specs=[
            pl.BlockSpec((NC, _RBP, DDEG), lambda i: (0, i, 0)),
            pl.BlockSpec((_RBP, m), lambda i: (i, 0)),
        ],
        out_specs=pl.BlockSpec((_RBP, m), lambda i: (i, 0)),
        out_shape=jax.ShapeDtypeStruct((NACC, m), jnp.float32),
    )(degp, h)


def _mid_body(degp_ref, hp_ref, aggp_ref, b_ref, w_ref, o_ref):
    deg = degp_ref[0, :, 0:1] + degp_ref[1, :, 0:1] + 1.0
    dis = lax.rsqrt(deg)
    tot = hp_ref[...] + aggp_ref[0] + aggp_ref[1]
    h2 = jnp.maximum(tot * dis + b_ref[...], 0.0)
    o_ref[...] = jnp.dot(h2, w_ref[...], preferred_element_type=jnp.float32) * dis


def _tc_mid(degp, hp1, aggp1, b1, w2):
    k, m = w2.shape
    return pl.pallas_call(
        _mid_body,
        grid=(_GRIDP,),
        in_specs=[
            pl.BlockSpec((NC, _RBP, DDEG), lambda i: (0, i, 0)),
            pl.BlockSpec((_RBP, k), lambda i: (i, 0)),
            pl.BlockSpec((NC, _RBP, k), lambda i: (0, i, 0)),
            pl.BlockSpec((1, k), lambda i: (0, 0)),
            pl.BlockSpec((k, m), lambda i: (0, 0)),
        ],
        out_specs=pl.BlockSpec((_RBP, m), lambda i: (i, 0)),
        out_shape=jax.ShapeDtypeStruct((NACC, m), jnp.float32),
    )(degp, hp1, aggp1, b1, w2)


def _fin_body(degp_ref, hp_ref, aggp_ref, b_ref, o_ref):
    deg = degp_ref[0, :, 0:1] + degp_ref[1, :, 0:1] + 1.0
    dis = lax.rsqrt(deg)
    tot = hp_ref[...] + aggp_ref[0] + aggp_ref[1]
    o_ref[...] = tot * dis + b_ref[...]


def _tc_fin(degp, hp2, aggp2, b2):
    m = hp2.shape[1]
    return pl.pallas_call(
        _fin_body,
        grid=(_GRID,),
        in_specs=[
            pl.BlockSpec((NC, _RB, DDEG), lambda i: (0, i, 0)),
            pl.BlockSpec((_RB, m), lambda i: (i, 0)),
            pl.BlockSpec((NC, _RB, m), lambda i: (0, i, 0)),
            pl.BlockSpec((1, m), lambda i: (0, 0)),
        ],
        out_specs=pl.BlockSpec((_RB, m), lambda i: (i, 0)),
        out_shape=jax.ShapeDtypeStruct((N, m), jnp.float32),
    )(degp, hp2, aggp2, b2)


def kernel(x, edge_index, W1, b1, W2, b2):
    xp = jnp.concatenate([x, jnp.zeros((NACC - N, x.shape[1]), jnp.float32)])
    src = edge_index[0].astype(jnp.int32)
    dst = edge_index[1].astype(jnp.int32)
    epad = NW * NCHUNK * CHUNK
    src_p = jnp.concatenate(
        [src, jnp.zeros((epad - E,), jnp.int32)]).reshape(NW, NCHUNK, CHUNK)
    dst_p = jnp.concatenate(
        [dst, jnp.full((epad - E,), N, jnp.int32)]).reshape(NW, NCHUNK, CHUNK)
    src_p = jnp.concatenate(
        [src_p, jnp.zeros((NW, 2, CHUNK), jnp.int32)], axis=1)
    dst_p = jnp.concatenate(
        [dst_p, jnp.full((NW, 2, CHUNK), N, jnp.int32)], axis=1)

    ones = jnp.ones((CHUNK, DDEG), jnp.float32)
    zeros_deg = jnp.zeros((ZROWS, DDEG), jnp.float32)
    zeros64 = jnp.zeros((ZB, 64), jnp.float32)
    zeros32 = jnp.zeros((ZB, 32), jnp.float32)

    degp = _deg_call(dst_p, ones, zeros_deg)
    h1 = _tc_matmul(xp, W1)
    hp1 = _tc_scale(degp, h1)
    aggp1 = _agg64(hp1, src_p, dst_p, zeros64)
    hp2 = _tc_mid(degp, hp1, aggp1, b1.reshape(1, 64), W2)
    aggp2 = _agg32(hp2, src_p, dst_p, zeros32)
    return _tc_fin(degp, hp2, aggp2, b2.reshape(1, 32))


# speedup vs baseline: 2.7516x; 1.1625x over previous
import functools

import jax
import jax.numpy as jnp
from jax import lax
from jax.experimental import pallas as pl
from jax.experimental.pallas import tpu as pltpu
from jax.experimental.pallas import tpu_sc as plsc

N = 10000
E = 320000
NC = 2
NS = 16
NW = NC * NS
CHUNK = 128
NCHUNK = 80
NCTOT = NCHUNK + 2
NACC = 10240
ZROWS = NACC // NS
ZB = 160
DDEG = 16

_MESH = plsc.VectorSubcoreMesh(core_axis_name="c", subcore_axis_name="s")
_SC_PARAMS = pltpu.CompilerParams(use_tc_tiling_on_sc=False)


def _make_deg():
    @functools.partial(
        pl.kernel,
        out_type=jax.ShapeDtypeStruct((NC, NACC, DDEG), jnp.float32),
        mesh=_MESH,
        compiler_params=_SC_PARAMS,
        scratch_types=[
            pltpu.VMEM((NCTOT, CHUNK), jnp.int32),
            pltpu.VMEM((CHUNK, DDEG), jnp.float32),
            pltpu.VMEM((ZROWS, DDEG), jnp.float32),
            pltpu.VMEM_SHARED((NACC, DDEG), jnp.float32),
            pltpu.SemaphoreType.DMA,
        ],
    )
    def deg(dst_hbm, ones_hbm, zeros_hbm, out_hbm, dst_v, ones_v, zbuf_v, acc_sh, sem):
        cid = lax.axis_index("c")
        sid = lax.axis_index("s")
        wid = sid * NC + cid
        pltpu.sync_copy(zeros_hbm, zbuf_v)
        pltpu.sync_copy(zbuf_v, acc_sh.at[pl.ds(sid * ZROWS, ZROWS)])
        pltpu.sync_copy(ones_hbm, ones_v)
        pltpu.sync_copy(dst_hbm.at[wid], dst_v)
        plsc.subcore_barrier()

        def body(j, carry):
            pltpu.sync_copy(ones_v, acc_sh.at[dst_v.at[j]], add=True)
            return carry

        lax.fori_loop(0, NCHUNK, body, 0)
        plsc.subcore_barrier()
        pltpu.sync_copy(acc_sh.at[pl.ds(sid * ZROWS, ZROWS)], zbuf_v)
        pltpu.sync_copy(zbuf_v, out_hbm.at[cid, pl.ds(sid * ZROWS, ZROWS)])

    return deg


def _make_agg(d):
    @functools.partial(
        pl.kernel,
        out_type=jax.ShapeDtypeStruct((NC, NACC, d), jnp.float32),
        mesh=_MESH,
        compiler_params=_SC_PARAMS,
        scratch_types=[
            pltpu.VMEM((NCTOT, CHUNK), jnp.int32),
            pltpu.VMEM((NCTOT, CHUNK), jnp.int32),
            pltpu.VMEM((CHUNK, d), jnp.float32),
            pltpu.VMEM((CHUNK, d), jnp.float32),
            pltpu.VMEM((ZB, d), jnp.float32),
            pltpu.VMEM_SHARED((NACC, d), jnp.float32),
            pltpu.VMEM_SHARED((NACC, d), jnp.float32),
            pltpu.SemaphoreType.DMA,
            pltpu.SemaphoreType.DMA,
        ],
    )
    def agg(hp_hbm, src_hbm, dst_hbm, zeros_hbm, out_hbm,
            src_v, dst_v, rows0, rows1, zbuf_v, table_sh, acc_sh, sem0, sem1):
        cid = lax.axis_index("c")
        sid = lax.axis_index("s")
        wid = sid * NC + cid
        pltpu.sync_copy(hp_hbm.at[pl.ds(sid * ZROWS, ZROWS)],
                        table_sh.at[pl.ds(sid * ZROWS, ZROWS)])
        pltpu.sync_copy(zeros_hbm, zbuf_v)
        for p in range(ZROWS // ZB):
            pltpu.sync_copy(zbuf_v, acc_sh.at[pl.ds(sid * ZROWS + p * ZB, ZB)])
        pltpu.sync_copy(src_hbm.at[wid], src_v)
        pltpu.sync_copy(dst_hbm.at[wid], dst_v)
        plsc.subcore_barrier()

        pltpu.async_copy(table_sh.at[src_v.at[0]], rows0, sem0)
        pltpu.async_copy(table_sh.at[src_v.at[1]], rows1, sem1)

        def body(i, carry):
            j = 2 * i
            pltpu.make_async_copy(table_sh.at[src_v.at[j]], rows0, sem0).wait()
            pltpu.sync_copy(rows0, acc_sh.at[dst_v.at[j]], add=True)
            pltpu.async_copy(table_sh.at[src_v.at[j + 2]], rows0, sem0)
            pltpu.make_async_copy(table_sh.at[src_v.at[j + 1]], rows1, sem1).wait()
            pltpu.sync_copy(rows1, acc_sh.at[dst_v.at[j + 1]], add=True)
            pltpu.async_copy(table_sh.at[src_v.at[j + 3]], rows1, sem1)
            return carry

        lax.fori_loop(0, NCHUNK // 2, body, 0)
        pltpu.make_async_copy(table_sh.at[src_v.at[NCHUNK]], rows0, sem0).wait()
        pltpu.make_async_copy(table_sh.at[src_v.at[NCHUNK + 1]], rows1, sem1).wait()
        plsc.subcore_barrier()
        for p in range(ZROWS // ZB):
            pltpu.sync_copy(acc_sh.at[pl.ds(sid * ZROWS + p * ZB, ZB)], zbuf_v)
            pltpu.sync_copy(zbuf_v,
                            out_hbm.at[cid, pl.ds(sid * ZROWS + p * ZB, ZB)])

    return agg


_deg_call = _make_deg()
_agg64 = _make_agg(64)
_agg32 = _make_agg(32)

_RB = 2000
_GRID = N // _RB
_RBP = 1280
_GRIDP = NACC // _RBP


def _mm_body(x_ref, w_ref, o_ref):
    o_ref[...] = jnp.dot(x_ref[...], w_ref[...],
                         preferred_element_type=jnp.float32)


def _tc_matmul(x, w):
    k, m = x.shape[1], w.shape[1]
    return pl.pallas_call(
        _mm_body,
        grid=(_GRIDP,),
        in_specs=[
            pl.BlockSpec((_RBP, k), lambda i: (i, 0)),
            pl.BlockSpec((k, m), lambda i: (0, 0)),
        ],
        out_specs=pl.BlockSpec((_RBP, m), lambda i: (i, 0)),
        out_shape=jax.ShapeDtypeStruct((NACC, m), jnp.float32),
    )(x, w)


def _scale_body(degp_ref, h_ref, o_ref):
    deg = degp_ref[0, :, 0:1] + degp_ref[1, :, 0:1] + 1.0
    o_ref[...] = h_ref[...] * lax.rsqrt(deg)


def _tc_scale(degp, h):
    m = h.shape[1]
    return pl.pallas_call(
        _scale_body,
        grid=(_GRIDP,),
        in_specs=[
            pl.BlockSpec((NC, _RBP, DDEG), lambda i: (0, i, 0)),
            pl.BlockSpec((_RBP, m), lambda i: (i, 0)),
        ],
        out_specs=pl.BlockSpec((_RBP, m), lambda i: (i, 0)),
        out_shape=jax.ShapeDtypeStruct((NACC, m), jnp.float32),
    )(degp, h)


def _mid_body(degp_ref, hp_ref, aggp_ref, b_ref, w_ref, o_ref):
    deg = degp_ref[0, :, 0:1] + degp_ref[1, :, 0:1] + 1.0
    dis = lax.rsqrt(deg)
    tot = hp_ref[...] + aggp_ref[0] + aggp_ref[1]
    h2 = jnp.maximum(tot * dis + b_ref[...], 0.0)
    o_ref[...] = jnp.dot(h2, w_ref[...], preferred_element_type=jnp.float32) * dis


def _tc_mid(degp, hp1, aggp1, b1, w2):
    k, m = w2.shape
    return pl.pallas_call(
        _mid_body,
        grid=(_GRIDP,),
        in_specs=[
            pl.BlockSpec((NC, _RBP, DDEG), lambda i: (0, i, 0)),
            pl.BlockSpec((_RBP, k), lambda i: (i, 0)),
            pl.BlockSpec((NC, _RBP, k), lambda i: (0, i, 0)),
            pl.BlockSpec((1, k), lambda i: (0, 0)),
            pl.BlockSpec((k, m), lambda i: (0, 0)),
        ],
        out_specs=pl.BlockSpec((_RBP, m), lambda i: (i, 0)),
        out_shape=jax.ShapeDtypeStruct((NACC, m), jnp.float32),
    )(degp, hp1, aggp1, b1, w2)


def _fin_body(degp_ref, hp_ref, aggp_ref, b_ref, o_ref):
    deg = degp_ref[0, :, 0:1] + degp_ref[1, :, 0:1] + 1.0
    dis = lax.rsqrt(deg)
    tot = hp_ref[...] + aggp_ref[0] + aggp_ref[1]
    o_ref[...] = tot * dis + b_ref[...]


def _tc_fin(degp, hp2, aggp2, b2):
    m = hp2.shape[1]
    return pl.pallas_call(
        _fin_body,
        grid=(_GRID,),
        in_specs=[
            pl.BlockSpec((NC, _RB, DDEG), lambda i: (0, i, 0)),
            pl.BlockSpec((_RB, m), lambda i: (i, 0)),
            pl.BlockSpec((NC, _RB, m), lambda i: (0, i, 0)),
            pl.BlockSpec((1, m), lambda i: (0, 0)),
        ],
        out_specs=pl.BlockSpec((_RB, m), lambda i: (i, 0)),
        out_shape=jax.ShapeDtypeStruct((N, m), jnp.float32),
    )(degp, hp2, aggp2, b2)


def kernel(x, edge_index, W1, b1, W2, b2):
    xp = jnp.concatenate([x, jnp.zeros((NACC - N, x.shape[1]), jnp.float32)])
    src = edge_index[0].astype(jnp.int32)
    dst = edge_index[1].astype(jnp.int32)
    epad = NW * NCHUNK * CHUNK
    src_p = jnp.concatenate(
        [src, jnp.zeros((epad - E,), jnp.int32)]).reshape(NW, NCHUNK, CHUNK)
    dst_p = jnp.concatenate(
        [dst, jnp.full((epad - E,), N, jnp.int32)]).reshape(NW, NCHUNK, CHUNK)
    src_p = jnp.concatenate(
        [src_p, jnp.zeros((NW, 2, CHUNK), jnp.int32)], axis=1)
    dst_p = jnp.concatenate(
        [dst_p, jnp.full((NW, 2, CHUNK), N, jnp.int32)], axis=1)

    ones = jnp.ones((CHUNK, DDEG), jnp.float32)
    zeros_deg = jnp.zeros((ZROWS, DDEG), jnp.float32)
    zeros64 = jnp.zeros((ZB, 64), jnp.float32)
    zeros32 = jnp.zeros((ZB, 32), jnp.float32)

    degp = _deg_call(dst_p, ones, zeros_deg)
    h1 = _tc_matmul(xp, W1)
    hp1 = _tc_scale(degp, h1)
    aggp1 = _agg64(hp1, src_p, dst_p, zeros64)
    hp2 = _tc_mid(degp, hp1, aggp1, b1.reshape(1, 64), W2)
    aggp2 = _agg32(hp2, src_p, dst_p, zeros32)
    return _tc_fin(degp, hp2, aggp2, b2.reshape(1, 32))
